# Initial kernel scaffold; baseline (speedup 1.0000x reference)
import jax
import jax.numpy as jnp
from jax.experimental import pallas as pl


def kernel(inputIDs, sequenceIDs, word_table, seq_table, gamma, beta):
    raise NotImplementedError("write your pallas kernel here")


# speedup vs baseline: 2.4262x; 2.4262x over previous
import functools
import math

import jax
import jax.numpy as jnp
import numpy as np
from jax import lax
from jax.experimental import pallas as pl
from jax.experimental.pallas import tpu as pltpu
from jax.experimental.pallas import tpu_sc as plsc

_HIDDEN = 768
_NC, _NS = 2, 16
_NW = _NC * _NS
_CH = 64


def _pe_row(seq_len: int, hidden: int) -> np.ndarray:
    norm = np.exp(np.arange(0, hidden, 2, dtype=np.float64)
                  * (-(math.log(10000.0) / hidden)))
    row = np.zeros((hidden,), dtype=np.float64)
    row[0::2] = np.sin(seq_len * norm)
    row[1::2] = np.cos(seq_len * norm)
    return row.astype(np.float32)


def _sc_gather(table, idx3):
    n_ch = idx3.shape[1]
    n = _NW * n_ch * _CH
    b_per_w = n_ch * _CH
    mesh = plsc.VectorSubcoreMesh(core_axis_name="c", subcore_axis_name="s")

    @functools.partial(
        pl.kernel,
        mesh=mesh,
        out_type=jax.ShapeDtypeStruct((n, _HIDDEN), jnp.float32),
        scratch_types=[
            pltpu.VMEM((n_ch, _CH), jnp.int32),
            pltpu.VMEM((_CH, _HIDDEN), jnp.float32),
            pltpu.VMEM((_CH, _HIDDEN), jnp.float32),
            pltpu.SemaphoreType.DMA,
            pltpu.SemaphoreType.DMA,
        ],
    )
    def k(table_hbm, idx_hbm, out_hbm, idx_v, buf0, buf1, sem0, sem1):
        wid = lax.axis_index("s") * _NC + lax.axis_index("c")
        base = wid * b_per_w
        pltpu.sync_copy(idx_hbm.at[wid], idx_v)
        bufs = (buf0, buf1)
        sems = (sem0, sem1)
        cps = [pltpu.async_copy(table_hbm.at[idx_v.at[i]], bufs[i], sems[i])
               for i in range(min(2, n_ch))]
        for i in range(n_ch):
            cps[i].wait()
            pltpu.sync_copy(bufs[i % 2], out_hbm.at[pl.ds(base + i * _CH, _CH)])
            if i + 2 < n_ch:
                cps.append(pltpu.async_copy(
                    table_hbm.at[idx_v.at[i + 2]], bufs[i % 2], sems[i % 2]))

    return k(table, idx3)


def _tc_layernorm(rows, seqf, bias0, dbias, gamma, beta):
    n = rows.shape[0]
    br = 1024
    grid = (n // br,)

    def body(rows_ref, seqf_ref, b0_ref, db_ref, g_ref, be_ref, out_ref):
        x = rows_ref[...]
        s = seqf_ref[...]
        x = x + b0_ref[...] + s * db_ref[...]
        mean = jnp.mean(x, axis=-1, keepdims=True)
        xc = x - mean
        var = jnp.mean(xc * xc, axis=-1, keepdims=True)
        rstd = lax.rsqrt(var + 1e-12)
        out_ref[...] = g_ref[...] * (xc * rstd) + be_ref[...]

    return pl.pallas_call(
        body,
        grid=grid,
        in_specs=[
            pl.BlockSpec((br, _HIDDEN), lambda i: (i, 0)),
            pl.BlockSpec((br, 1), lambda i: (i, 0)),
            pl.BlockSpec((1, _HIDDEN), lambda i: (0, 0)),
            pl.BlockSpec((1, _HIDDEN), lambda i: (0, 0)),
            pl.BlockSpec((1, _HIDDEN), lambda i: (0, 0)),
            pl.BlockSpec((1, _HIDDEN), lambda i: (0, 0)),
        ],
        out_specs=pl.BlockSpec((br, _HIDDEN), lambda i: (i, 0)),
        out_shape=jax.ShapeDtypeStruct((n, _HIDDEN), jnp.float32),
    )(rows, seqf, bias0, dbias, gamma, beta)


def kernel(inputIDs, sequenceIDs, word_table, seq_table, gamma, beta):
    b, l = inputIDs.shape
    n = b * l
    idx3 = inputIDs.reshape(_NW, n // (_NW * _CH), _CH).astype(jnp.int32)

    gathered = _sc_gather(word_table, idx3)

    pe = jnp.asarray(_pe_row(l, _HIDDEN))
    bias0 = (seq_table[0] + pe).reshape(1, _HIDDEN)
    dbias = (seq_table[1] - seq_table[0]).reshape(1, _HIDDEN)
    seqf = sequenceIDs.reshape(n, 1).astype(jnp.float32)

    out = _tc_layernorm(gathered, seqf, bias0, dbias,
                        gamma.reshape(1, _HIDDEN), beta.reshape(1, _HIDDEN))
    return out.reshape(b, l, _HIDDEN)
